# Initial kernel scaffold; baseline (speedup 1.0000x reference)
import jax
import jax.numpy as jnp
from jax.experimental import pallas as pl


def kernel(inputs, targets, lut):
    raise NotImplementedError("write your pallas kernel here")


# speedup vs baseline: 1.0658x; 1.0658x over previous
import functools

import jax
import jax.numpy as jnp
from jax import lax
from jax.experimental import pallas as pl
from jax.experimental.pallas import tpu as pltpu
from jax.experimental.pallas import tpu_sc as plsc

_SCALAR = 20.0


def _sc_gather(lut, targets):
    k_, d_ = lut.shape
    b_ = targets.shape[0]
    info = plsc.get_sparse_core_info()
    nw = info.num_cores * info.num_subcores
    b_per_w = b_ // nw
    mesh = plsc.VectorSubcoreMesh(core_axis_name="c", subcore_axis_name="s")

    @functools.partial(
        pl.kernel,
        mesh=mesh,
        compiler_params=pltpu.CompilerParams(use_tc_tiling_on_sc=False),
        out_type=jax.ShapeDtypeStruct((b_, d_), jnp.float32),
        scratch_types=[
            pltpu.VMEM((b_per_w,), jnp.int32),
            pltpu.VMEM((b_per_w, d_), jnp.float32),
            pltpu.SemaphoreType.DMA,
        ],
    )
    def gather_kernel(table_hbm, idx_hbm, out_hbm, idx_v, rows_v, sem):
        wid = lax.axis_index("s") * info.num_cores + lax.axis_index("c")
        base = wid * b_per_w
        pltpu.sync_copy(idx_hbm.at[pl.ds(base, b_per_w)], idx_v)
        pltpu.async_copy(table_hbm.at[idx_v], rows_v, sem).wait()
        pltpu.sync_copy(rows_v, out_hbm.at[pl.ds(base, b_per_w)])

    return gather_kernel(lut, targets)


def _lse_loss_body(x_ref, lut_ref, rows_ref, out_ref, m_ref, s_ref):
    kstep = pl.program_id(0)
    nk = pl.num_programs(0)

    @pl.when(kstep == 0)
    def _init():
        m_ref[...] = jnp.full_like(m_ref, -jnp.inf)
        s_ref[...] = jnp.zeros_like(s_ref)

    x = x_ref[...]
    t = lax.dot_general(
        x, lut_ref[...],
        dimension_numbers=(((1,), (1,)), ((), ())),
        preferred_element_type=jnp.float32,
    ) * _SCALAR
    m_old = m_ref[...]
    m_new = jnp.maximum(m_old, jnp.max(t, axis=1, keepdims=True))
    p_sum = jnp.sum(jnp.exp(t - m_new), axis=1, keepdims=True)
    s_ref[...] = s_ref[...] * jnp.exp(m_old - m_new) + p_sum
    m_ref[...] = m_new

    @pl.when(kstep == nk - 1)
    def _finish():
        picked = _SCALAR * jnp.sum(x * rows_ref[...], axis=1, keepdims=True)
        lse = m_ref[...] + jnp.log(s_ref[...])
        out_ref[...] = jnp.sum(lse - picked, axis=0, keepdims=True) / x.shape[0]


def _tc_lse_loss(inputs, lut, rows, tk=2000):
    b_, d_ = inputs.shape
    k_ = lut.shape[0]
    nk = k_ // tk
    out = pl.pallas_call(
        _lse_loss_body,
        grid=(nk,),
        in_specs=[
            pl.BlockSpec((b_, d_), lambda k: (0, 0)),
            pl.BlockSpec((tk, d_), lambda k: (k, 0)),
            pl.BlockSpec((b_, d_), lambda k: (0, 0)),
        ],
        out_specs=pl.BlockSpec((1, 1), lambda k: (0, 0)),
        out_shape=jax.ShapeDtypeStruct((1, 1), jnp.float32),
        scratch_shapes=[
            pltpu.VMEM((b_, 1), jnp.float32),
            pltpu.VMEM((b_, 1), jnp.float32),
        ],
    )(inputs, lut, rows)
    return out[0, 0]


def kernel(inputs, targets, lut):
    rows = _sc_gather(lut, targets)
    return _tc_lse_loss(inputs, lut, rows)


# speedup vs baseline: 1.0713x; 1.0051x over previous
import functools

import jax
import jax.numpy as jnp
from jax import lax
from jax.experimental import pallas as pl
from jax.experimental.pallas import tpu as pltpu
from jax.experimental.pallas import tpu_sc as plsc

_SCALAR = 20.0


def _sc_gather(lut, targets):
    k_, d_ = lut.shape
    b_ = targets.shape[0]
    info = plsc.get_sparse_core_info()
    nw = info.num_cores * info.num_subcores
    b_per_w = b_ // nw
    mesh = plsc.VectorSubcoreMesh(core_axis_name="c", subcore_axis_name="s")

    @functools.partial(
        pl.kernel,
        mesh=mesh,
        compiler_params=pltpu.CompilerParams(use_tc_tiling_on_sc=False),
        out_type=jax.ShapeDtypeStruct((b_, d_), jnp.float32),
        scratch_types=[
            pltpu.VMEM((b_per_w,), jnp.int32),
            pltpu.VMEM((b_per_w, d_), jnp.float32),
            pltpu.SemaphoreType.DMA,
        ],
    )
    def gather_kernel(table_hbm, idx_hbm, out_hbm, idx_v, rows_v, sem):
        wid = lax.axis_index("s") * info.num_cores + lax.axis_index("c")
        base = wid * b_per_w
        pltpu.sync_copy(idx_hbm.at[pl.ds(base, b_per_w)], idx_v)
        pltpu.async_copy(table_hbm.at[idx_v], rows_v, sem).wait()
        pltpu.sync_copy(rows_v, out_hbm.at[pl.ds(base, b_per_w)])

    return gather_kernel(lut, targets)


def _lse_loss_body(x_ref, lut_ref, rows_ref, out_ref, m_ref, s_ref):
    kstep = pl.program_id(0)
    nk = pl.num_programs(0)

    @pl.when(kstep == 0)
    def _init():
        m_ref[...] = jnp.full_like(m_ref, -jnp.inf)
        s_ref[...] = jnp.zeros_like(s_ref)

    x = x_ref[...]
    t = lax.dot_general(
        x.astype(jnp.bfloat16), lut_ref[...].astype(jnp.bfloat16),
        dimension_numbers=(((1,), (1,)), ((), ())),
        preferred_element_type=jnp.float32,
    ) * _SCALAR
    m_old = m_ref[...]
    m_new = jnp.maximum(m_old, jnp.max(t, axis=1, keepdims=True))
    p_sum = jnp.sum(jnp.exp(t - m_new), axis=1, keepdims=True)
    s_ref[...] = s_ref[...] * jnp.exp(m_old - m_new) + p_sum
    m_ref[...] = m_new

    @pl.when(kstep == nk - 1)
    def _finish():
        picked = _SCALAR * jnp.sum(x * rows_ref[...], axis=1, keepdims=True)
        lse = m_ref[...] + jnp.log(s_ref[...])
        out_ref[...] = jnp.sum(lse - picked, axis=0, keepdims=True) / x.shape[0]


def _tc_lse_loss(inputs, lut, rows, tk=2000):
    b_, d_ = inputs.shape
    k_ = lut.shape[0]
    nk = k_ // tk
    out = pl.pallas_call(
        _lse_loss_body,
        grid=(nk,),
        in_specs=[
            pl.BlockSpec((b_, d_), lambda k: (0, 0)),
            pl.BlockSpec((tk, d_), lambda k: (k, 0)),
            pl.BlockSpec((b_, d_), lambda k: (0, 0)),
        ],
        out_specs=pl.BlockSpec((1, 1), lambda k: (0, 0)),
        out_shape=jax.ShapeDtypeStruct((1, 1), jnp.float32),
        scratch_shapes=[
            pltpu.VMEM((b_, 1), jnp.float32),
            pltpu.VMEM((b_, 1), jnp.float32),
        ],
    )(inputs, lut, rows)
    return out[0, 0]


def kernel(inputs, targets, lut):
    rows = _sc_gather(lut, targets)
    return _tc_lse_loss(inputs, lut, rows)


# speedup vs baseline: 1.1831x; 1.1043x over previous
import functools

import jax
import jax.numpy as jnp
from jax import lax
from jax.experimental import pallas as pl
from jax.experimental.pallas import tpu as pltpu
from jax.experimental.pallas import tpu_sc as plsc

_SCALAR = 20.0


def _sc_gather(lut, targets):
    k_, d_ = lut.shape
    b_ = targets.shape[0]
    info = plsc.get_sparse_core_info()
    nw = info.num_cores * info.num_subcores
    b_per_w = b_ // nw
    mesh = plsc.VectorSubcoreMesh(core_axis_name="c", subcore_axis_name="s")

    @functools.partial(
        pl.kernel,
        mesh=mesh,
        compiler_params=pltpu.CompilerParams(use_tc_tiling_on_sc=False),
        out_type=jax.ShapeDtypeStruct((b_, d_), jnp.float32),
        scratch_types=[
            pltpu.VMEM((b_per_w,), jnp.int32),
            pltpu.VMEM((b_per_w, d_), jnp.float32),
            pltpu.SemaphoreType.DMA,
        ],
    )
    def gather_kernel(table_hbm, idx_hbm, out_hbm, idx_v, rows_v, sem):
        wid = lax.axis_index("s") * info.num_cores + lax.axis_index("c")
        base = wid * b_per_w
        pltpu.sync_copy(idx_hbm.at[pl.ds(base, b_per_w)], idx_v)
        pltpu.async_copy(table_hbm.at[idx_v], rows_v, sem).wait()
        pltpu.sync_copy(rows_v, out_hbm.at[pl.ds(base, b_per_w)])

    return gather_kernel(lut, targets)


def _lse_loss_body(x_ref, lut_ref, rows_ref, out_ref, m_ref, s_ref):
    kstep = pl.program_id(0)
    nk = pl.num_programs(0)

    @pl.when(kstep == 0)
    def _init():
        m_ref[...] = jnp.full_like(m_ref, -jnp.inf)
        s_ref[...] = jnp.zeros_like(s_ref)

    x = x_ref[...]
    xb = (x * _SCALAR).astype(jnp.bfloat16)
    t = lax.dot_general(
        xb, lut_ref[...].astype(jnp.bfloat16),
        dimension_numbers=(((1,), (1,)), ((), ())),
        preferred_element_type=jnp.float32,
    ).astype(jnp.bfloat16)
    m_old = m_ref[...]
    m_new = jnp.maximum(m_old, jnp.max(t, axis=1, keepdims=True).astype(jnp.float32))
    p = jnp.exp(t - m_new.astype(jnp.bfloat16))
    ones = jnp.ones((t.shape[1], 1), dtype=jnp.bfloat16)
    p_sum = lax.dot_general(
        p, ones,
        dimension_numbers=(((1,), (0,)), ((), ())),
        preferred_element_type=jnp.float32,
    )
    s_ref[...] = s_ref[...] * jnp.exp(m_old - m_new) + p_sum
    m_ref[...] = m_new

    @pl.when(kstep == nk - 1)
    def _finish():
        picked = _SCALAR * jnp.sum(x * rows_ref[...], axis=1, keepdims=True)
        lse = m_ref[...] + jnp.log(s_ref[...])
        out_ref[...] = jnp.sum(lse - picked, axis=0, keepdims=True) / x.shape[0]


def _tc_lse_loss(inputs, lut, rows, tk=4000):
    b_, d_ = inputs.shape
    k_ = lut.shape[0]
    nk = k_ // tk
    out = pl.pallas_call(
        _lse_loss_body,
        grid=(nk,),
        in_specs=[
            pl.BlockSpec((b_, d_), lambda k: (0, 0)),
            pl.BlockSpec((tk, d_), lambda k: (k, 0)),
            pl.BlockSpec((b_, d_), lambda k: (0, 0)),
        ],
        out_specs=pl.BlockSpec((1, 1), lambda k: (0, 0)),
        out_shape=jax.ShapeDtypeStruct((1, 1), jnp.float32),
        scratch_shapes=[
            pltpu.VMEM((b_, 1), jnp.float32),
            pltpu.VMEM((b_, 1), jnp.float32),
        ],
    )(inputs, lut, rows)
    return out[0, 0]


def kernel(inputs, targets, lut):
    rows = _sc_gather(lut, targets)
    return _tc_lse_loss(inputs, lut, rows)


# speedup vs baseline: 1.1872x; 1.0035x over previous
import functools

import jax
import jax.numpy as jnp
from jax import lax
from jax.experimental import pallas as pl
from jax.experimental.pallas import tpu as pltpu
from jax.experimental.pallas import tpu_sc as plsc

_SCALAR = 20.0


def _sc_gather(lut, targets):
    k_, d_ = lut.shape
    b_ = targets.shape[0]
    info = plsc.get_sparse_core_info()
    nw = info.num_cores * info.num_subcores
    b_per_w = b_ // nw
    mesh = plsc.VectorSubcoreMesh(core_axis_name="c", subcore_axis_name="s")

    @functools.partial(
        pl.kernel,
        mesh=mesh,
        compiler_params=pltpu.CompilerParams(use_tc_tiling_on_sc=False),
        out_type=jax.ShapeDtypeStruct((b_, d_), jnp.float32),
        scratch_types=[
            pltpu.VMEM((b_per_w,), jnp.int32),
            pltpu.VMEM((b_per_w, d_), jnp.float32),
            pltpu.SemaphoreType.DMA,
        ],
    )
    def gather_kernel(table_hbm, idx_hbm, out_hbm, idx_v, rows_v, sem):
        wid = lax.axis_index("s") * info.num_cores + lax.axis_index("c")
        base = wid * b_per_w
        pltpu.sync_copy(idx_hbm.at[pl.ds(base, b_per_w)], idx_v)
        pltpu.async_copy(table_hbm.at[idx_v], rows_v, sem).wait()
        pltpu.sync_copy(rows_v, out_hbm.at[pl.ds(base, b_per_w)])

    return gather_kernel(lut, targets)


def _lse_loss_body(x_ref, lut_ref, rows_ref, out_ref, m_ref, s_ref):
    kstep = pl.program_id(0)
    nk = pl.num_programs(0)

    @pl.when(kstep == 0)
    def _init():
        m_ref[...] = jnp.full_like(m_ref, -jnp.inf)
        s_ref[...] = jnp.zeros_like(s_ref)

    x = x_ref[...]
    xb = (x * _SCALAR).astype(jnp.bfloat16)
    t = lax.dot_general(
        xb, lut_ref[...].astype(jnp.bfloat16),
        dimension_numbers=(((1,), (1,)), ((), ())),
        preferred_element_type=jnp.float32,
    ).astype(jnp.bfloat16)
    m_old = m_ref[...]
    m_new = jnp.maximum(m_old, jnp.max(t, axis=1, keepdims=True).astype(jnp.float32))
    p = jnp.exp(t - m_new.astype(jnp.bfloat16))
    p_sum = jnp.sum(p, axis=1, keepdims=True).astype(jnp.float32)
    s_ref[...] = s_ref[...] * jnp.exp(m_old - m_new) + p_sum
    m_ref[...] = m_new

    @pl.when(kstep == nk - 1)
    def _finish():
        picked = _SCALAR * jnp.sum(x * rows_ref[...], axis=1, keepdims=True)
        lse = m_ref[...] + jnp.log(s_ref[...])
        out_ref[...] = jnp.sum(lse - picked, axis=0, keepdims=True) / x.shape[0]


def _tc_lse_loss(inputs, lut, rows, tk=4000):
    b_, d_ = inputs.shape
    k_ = lut.shape[0]
    nk = k_ // tk
    out = pl.pallas_call(
        _lse_loss_body,
        grid=(nk,),
        in_specs=[
            pl.BlockSpec((b_, d_), lambda k: (0, 0)),
            pl.BlockSpec((tk, d_), lambda k: (k, 0)),
            pl.BlockSpec((b_, d_), lambda k: (0, 0)),
        ],
        out_specs=pl.BlockSpec((1, 1), lambda k: (0, 0)),
        out_shape=jax.ShapeDtypeStruct((1, 1), jnp.float32),
        scratch_shapes=[
            pltpu.VMEM((b_, 1), jnp.float32),
            pltpu.VMEM((b_, 1), jnp.float32),
        ],
    )(inputs, lut, rows)
    return out[0, 0]


def kernel(inputs, targets, lut):
    rows = _sc_gather(lut, targets)
    return _tc_lse_loss(inputs, lut, rows)
